# baseline (speedup 1.0000x reference)
import functools

import numpy as np
import jax
import jax.numpy as jnp
from jax import lax
from jax.experimental import pallas as pl
from jax.experimental.pallas import tpu as pltpu
from jax.experimental.pallas import tpu_sc as plsc

_B, _C, _N, _K = 4, 128, 2048, 16
_DH, _DO = 64, 128
_W = 128
_EPS = 1e-5
_E = _B * _N * _K
_PTS = _B * _N
_RT = 256
_ET = 2048
_PT = _ET // _K
_NT = _E // _ET
_HI = lax.Precision.HIGHEST

_AT = np.kron(np.eye(_PT, dtype=np.float32), np.ones((_K, 1), np.float32))
_P = np.zeros((_W, _W), np.float32)
_P[_DH:, :_DH] = np.eye(_DH, dtype=np.float32)


def _dot(a, b, dims):
    return lax.dot_general(a, b, (dims, ((), ())), precision=_HI,
                           preferred_element_type=jnp.float32)


def _bdot(a, b, dims):
    return lax.dot_general(a.astype(jnp.bfloat16), b.astype(jnp.bfloat16),
                           (dims, ((), ())),
                           preferred_element_type=jnp.float32)


def _pre_body(x_ref, w_ref, bv_ref, yt_ref, xx_ref):
    x2 = x_ref[0]
    yt_ref[0] = _dot(x2, w_ref[...], ((0,), (1,))) + bv_ref[...]
    xx_ref[0] = jnp.sum(x2 * x2, axis=0, keepdims=True)


def _pre(x, w, bv, interpret=False):
    return pl.pallas_call(
        _pre_body,
        grid=(_B,),
        in_specs=[
            pl.BlockSpec((1, _C, _N), lambda b: (b, 0, 0)),
            pl.BlockSpec((_W, _C), lambda b: (0, 0)),
            pl.BlockSpec((1, _W), lambda b: (0, 0)),
        ],
        out_specs=[
            pl.BlockSpec((1, _N, _W), lambda b: (b, 0, 0)),
            pl.BlockSpec((1, 1, _N), lambda b: (b, 0, 0)),
        ],
        out_shape=[
            jax.ShapeDtypeStruct((_B, _N, _W), jnp.float32),
            jax.ShapeDtypeStruct((_B, 1, _N), jnp.float32),
        ],
        interpret=interpret,
    )(x, w, bv)


def _knn_body(xr_ref, x_ref, xxr_ref, idx_ref):
    b = pl.program_id(0)
    xr = xr_ref[0]
    x2 = x_ref[0]
    ip = _bdot(x2, xr, ((0,), (0,)))
    ones = jnp.ones((_C, 1), jnp.float32)
    cn = _dot(x2 * x2, ones, ((0,), (0,)))
    d = -cn - (-2.0 * ip) - xxr_ref[0]
    iota = lax.broadcasted_iota(jnp.int32, (_N, _RT), 0)
    off = b * _N
    rows = []
    for _ in range(_K):
        m = jnp.max(d, axis=0, keepdims=True)
        cand = jnp.where(d == m, iota, _N)
        i = jnp.min(cand, axis=0, keepdims=True)
        rows.append(i + off)
        d = jnp.where(iota == i, -jnp.inf, d)
    idx_ref[0] = jnp.concatenate(rows, axis=0)


def _knn(x, xx, interpret=False):
    return pl.pallas_call(
        _knn_body,
        grid=(_B, _N // _RT),
        in_specs=[
            pl.BlockSpec((1, _C, _RT), lambda b, t: (b, 0, t)),
            pl.BlockSpec((1, _C, _N), lambda b, t: (b, 0, 0)),
            pl.BlockSpec((1, 1, _RT), lambda b, t: (b, 0, t)),
        ],
        out_specs=pl.BlockSpec((1, _K, _RT), lambda b, t: (b, 0, t)),
        out_shape=jax.ShapeDtypeStruct((_B, _K, _N), jnp.int32),
        interpret=interpret,
    )(x, x, xx)


_NW = 32
_RPW = _E // _NW
_CH = 256
_NCH = _RPW // _CH


def _sc_gather(table, gidx):
    mesh = plsc.VectorSubcoreMesh(core_axis_name="c", subcore_axis_name="s")

    @functools.partial(
        pl.kernel,
        mesh=mesh,
        out_type=jax.ShapeDtypeStruct((_E, _W), jnp.float32),
        scratch_types=[
            pltpu.VMEM((_RPW,), jnp.int32),
            pltpu.VMEM((2, _CH, _W), jnp.float32),
            pltpu.SemaphoreType.DMA,
            pltpu.SemaphoreType.DMA,
        ],
    )
    def k(table_hbm, idx_hbm, out_hbm, idx_v, bufs, sem0, sem1):
        wid = lax.axis_index("s") * 2 + lax.axis_index("c")
        base = wid * _RPW
        pltpu.sync_copy(idx_hbm.at[pl.ds(base, _RPW)], idx_v)
        sems = (sem0, sem1)

        def start(i):
            return pltpu.async_copy(
                table_hbm.at[idx_v.at[pl.ds(i * _CH, _CH)]],
                bufs.at[i % 2], sems[i % 2])

        cps = {0: start(0)}
        for i in range(_NCH):
            if i + 1 < _NCH:
                cps[i + 1] = start(i + 1)
            cps[i].wait()
            pltpu.sync_copy(bufs.at[i % 2],
                            out_hbm.at[pl.ds(base + i * _CH, _CH)])

    return k(table, gidx)


def _stats_body(g_ref, yt_ref, at_ref, p_ref, o_ref):
    g = g_ref[...]
    ybp = _dot(yt_ref[...], p_ref[...], ((1,), (0,)))
    gs = _bdot(at_ref[...], g, ((0,), (0,)))
    o_ref[0] = jnp.concatenate([
        jnp.sum(g, axis=0, keepdims=True),
        jnp.sum(g * g, axis=0, keepdims=True),
        jnp.sum(gs * ybp, axis=0, keepdims=True),
        jnp.sum(ybp, axis=0, keepdims=True),
        jnp.sum(ybp * ybp, axis=0, keepdims=True),
    ], axis=0)


def _stats(g, yt, at, p, interpret=False):
    return pl.pallas_call(
        _stats_body,
        grid=(_NT,),
        in_specs=[
            pl.BlockSpec((_ET, _W), lambda t: (t, 0)),
            pl.BlockSpec((_PT, _W), lambda t: (t, 0)),
            pl.BlockSpec((_ET, _PT), lambda t: (0, 0)),
            pl.BlockSpec((_W, _W), lambda t: (0, 0)),
        ],
        out_specs=pl.BlockSpec((1, 5, _W), lambda t: (t, 0, 0)),
        out_shape=jax.ShapeDtypeStruct((_NT, 5, _W), jnp.float32),
        interpret=interpret,
    )(g, yt, at, p)


def _mid_body(g_ref, yt_ref, at_ref, p_ref, s1_ref, t1_ref, s_ref, sr_ref):
    ybp = _dot(yt_ref[...], p_ref[...], ((1,), (0,)))
    ybx = _bdot(at_ref[...], ybp, ((1,), (0,)))
    h = g_ref[...] + ybx
    r = jnp.maximum(h * s1_ref[...] + t1_ref[...], 0.0)
    s_ref[0] = _bdot(r, r, ((0,), (0,)))
    sr_ref[0] = jnp.sum(r, axis=0, keepdims=True)


def _mid(g, yt, at, p, s1, t1, interpret=False):
    return pl.pallas_call(
        _mid_body,
        grid=(_NT,),
        in_specs=[
            pl.BlockSpec((_ET, _W), lambda t: (t, 0)),
            pl.BlockSpec((_PT, _W), lambda t: (t, 0)),
            pl.BlockSpec((_ET, _PT), lambda t: (0, 0)),
            pl.BlockSpec((_W, _W), lambda t: (0, 0)),
            pl.BlockSpec((1, _W), lambda t: (0, 0)),
            pl.BlockSpec((1, _W), lambda t: (0, 0)),
        ],
        out_specs=[
            pl.BlockSpec((1, _W, _W), lambda t: (t, 0, 0)),
            pl.BlockSpec((1, 1, _W), lambda t: (t, 0, 0)),
        ],
        out_shape=[
            jax.ShapeDtypeStruct((_NT, _W, _W), jnp.float32),
            jax.ShapeDtypeStruct((_NT, 1, _W), jnp.float32),
        ],
        interpret=interpret,
    )(g, yt, at, p, s1, t1)


def _bn2_body(s_ref, sr_ref, w2_ref, b2_ref, g2_ref, be2_ref,
              sc_ref, sh_ref):
    w2 = w2_ref[...]
    m = _dot(w2, s_ref[...], ((1,), (0,)))
    diag = jnp.sum(m * w2, axis=1, keepdims=True)
    wsr = _dot(w2, sr_ref[...], ((1,), (1,)))
    b2 = b2_ref[...]
    inv_e = 1.0 / _E
    mean2 = wsr * inv_e + b2
    ex2 = diag * inv_e + 2.0 * b2 * wsr * inv_e + b2 * b2
    var2 = ex2 - mean2 * mean2
    sc2 = g2_ref[...] / jnp.sqrt(var2 + _EPS)
    sc_ref[...] = sc2
    sh_ref[...] = be2_ref[...] - mean2 * sc2 + sc2 * b2


def _bn2(s, sr, w2, b2c, g2c, be2c, interpret=False):
    full = lambda shp: pl.BlockSpec(shp, lambda: (0,) * len(shp))
    return pl.pallas_call(
        _bn2_body,
        grid=(),
        in_specs=[full((_DH, _DH)), full((1, _DH)), full((_DO, _DH)),
                  full((_DO, 1)), full((_DO, 1)), full((_DO, 1))],
        out_specs=[full((_DO, 1)), full((_DO, 1))],
        out_shape=[jax.ShapeDtypeStruct((_DO, 1), jnp.float32),
                   jax.ShapeDtypeStruct((_DO, 1), jnp.float32)],
        interpret=interpret,
    )(s, sr, w2, b2c, g2c, be2c)


def _final_body(g_ref, yt_ref, at_ref, p_ref, w2_ref, s1_ref, t1_ref,
                s2_ref, t2_ref, o_ref):
    ybp = _dot(yt_ref[...], p_ref[...], ((1,), (0,)))
    ybx = _bdot(at_ref[...], ybp, ((1,), (0,)))
    h = g_ref[...] + ybx
    r = jnp.maximum(h * s1_ref[...] + t1_ref[...], 0.0)
    h2 = _bdot(w2_ref[...], r, ((1,), (1,)))
    o_ref[0] = jnp.maximum(h2 * s2_ref[...] + t2_ref[...], 0.0)


def _final(g, yt, at, p, w2p, s1, t1, s2, t2, interpret=False):
    return pl.pallas_call(
        _final_body,
        grid=(_NT,),
        in_specs=[
            pl.BlockSpec((_ET, _W), lambda t: (t, 0)),
            pl.BlockSpec((_PT, _W), lambda t: (t, 0)),
            pl.BlockSpec((_ET, _PT), lambda t: (0, 0)),
            pl.BlockSpec((_W, _W), lambda t: (0, 0)),
            pl.BlockSpec((_DO, _W), lambda t: (0, 0)),
            pl.BlockSpec((1, _W), lambda t: (0, 0)),
            pl.BlockSpec((1, _W), lambda t: (0, 0)),
            pl.BlockSpec((_DO, 1), lambda t: (0, 0)),
            pl.BlockSpec((_DO, 1), lambda t: (0, 0)),
        ],
        out_specs=pl.BlockSpec((1, _DO, _ET),
                               lambda t: (t // (_N * _K // _ET), 0,
                                          t % (_N * _K // _ET))),
        out_shape=jax.ShapeDtypeStruct((_B, _DO, _N * _K), jnp.float32),
        interpret=interpret,
    )(g, yt, at, p, w2p, s1, t1, s2, t2)


def kernel(x, W1, b1, g1, be1, W2, b2, g2, be2):
    wa = W1[:, :_C]
    w = jnp.concatenate([wa, W1[:, _C:] - wa], axis=0)
    bv = jnp.concatenate([jnp.zeros((_DH,), jnp.float32), b1])[None, :]
    at = jnp.asarray(_AT)
    p = jnp.asarray(_P)

    yt, xx = _pre(x, w, bv)
    idx = _knn(x, xx)

    table = yt.reshape(_PTS, _W)
    g = _sc_gather(table, jnp.swapaxes(idx, 1, 2).reshape(_E))

    st = jnp.sum(_stats(g, table, at, p), axis=0)[:, :_DH]
    mean1 = (st[0] + float(_K) * st[3]) / _E
    ex2 = (st[1] + 2.0 * st[2] + float(_K) * st[4]) / _E
    var1 = ex2 - mean1 * mean1
    sc1 = g1 / jnp.sqrt(var1 + _EPS)
    sh1 = be1 - mean1 * sc1
    zpad = jnp.zeros((_DH,), jnp.float32)
    s1 = jnp.concatenate([sc1, zpad])[None, :]
    t1 = jnp.concatenate([sh1, zpad])[None, :]

    sp, srp = _mid(g, table, at, p, s1, t1)
    s = jnp.sum(sp, axis=0)[:_DH, :_DH]
    sr = jnp.sum(srp, axis=0)[:, :_DH]

    sc2, sh2 = _bn2(s, sr, W2, b2[:, None], g2[:, None], be2[:, None])

    w2p = jnp.concatenate([W2, jnp.zeros((_DO, _DH), jnp.float32)], axis=1)
    out = _final(g, table, at, p, w2p, s1, t1, sc2, sh2)
    return out.reshape(_B, _DO, _N, _K)


# speedup vs baseline: 1.0480x; 1.0480x over previous
import functools

import numpy as np
import jax
import jax.numpy as jnp
from jax import lax
from jax.experimental import pallas as pl
from jax.experimental.pallas import tpu as pltpu
from jax.experimental.pallas import tpu_sc as plsc

_B, _C, _N, _K = 4, 128, 2048, 16
_DH, _DO = 64, 128
_W = 128
_EPS = 1e-5
_E = _B * _N * _K
_PTS = _B * _N
_RT = 256
_ET = 2048
_PT = _ET // _K
_NT = _E // _ET
_HI = lax.Precision.HIGHEST

_AT = np.kron(np.eye(_PT, dtype=np.float32), np.ones((_K, 1), np.float32))
_P = np.zeros((_W, _W), np.float32)
_P[_DH:, :_DH] = np.eye(_DH, dtype=np.float32)


def _dot(a, b, dims):
    return lax.dot_general(a, b, (dims, ((), ())), precision=_HI,
                           preferred_element_type=jnp.float32)


def _bdot(a, b, dims):
    return lax.dot_general(a.astype(jnp.bfloat16), b.astype(jnp.bfloat16),
                           (dims, ((), ())),
                           preferred_element_type=jnp.float32)


def _pre_body(x_ref, w_ref, bv_ref, yt_ref, xx_ref):
    x2 = x_ref[0]
    yt_ref[0] = _dot(x2, w_ref[...], ((0,), (1,))) + bv_ref[...]
    xx_ref[0] = jnp.sum(x2 * x2, axis=0, keepdims=True)


def _pre(x, w, bv, interpret=False):
    return pl.pallas_call(
        _pre_body,
        grid=(_B,),
        in_specs=[
            pl.BlockSpec((1, _C, _N), lambda b: (b, 0, 0)),
            pl.BlockSpec((_W, _C), lambda b: (0, 0)),
            pl.BlockSpec((1, _W), lambda b: (0, 0)),
        ],
        out_specs=[
            pl.BlockSpec((1, _N, _W), lambda b: (b, 0, 0)),
            pl.BlockSpec((1, 1, _N), lambda b: (b, 0, 0)),
        ],
        out_shape=[
            jax.ShapeDtypeStruct((_B, _N, _W), jnp.float32),
            jax.ShapeDtypeStruct((_B, 1, _N), jnp.float32),
        ],
        interpret=interpret,
    )(x, w, bv)


def _knn_body(xr_ref, x_ref, xx_ref, idx_ref):
    b = pl.program_id(0)
    xr = xr_ref[0]
    x2 = x_ref[0]
    ip = _bdot(xr, x2, ((0,), (0,)))
    ones = jnp.ones((_C, 1), jnp.float32)
    rn = _dot(xr * xr, ones, ((0,), (0,)))
    d = -xx_ref[0] - (-2.0 * ip) - rn
    iota = lax.broadcasted_iota(jnp.int32, (_RT, _N), 1)
    off = b * _N
    cols = []
    for _ in range(_K):
        m = jnp.max(d, axis=1, keepdims=True)
        cand = jnp.where(d == m, iota, _N)
        i = jnp.min(cand, axis=1, keepdims=True)
        cols.append(i + off)
        d = jnp.where(iota == i, -jnp.inf, d)
    idx_ref[0] = jnp.concatenate(cols, axis=1)


def _knn(x, xx, interpret=False):
    return pl.pallas_call(
        _knn_body,
        grid=(_B, _N // _RT),
        in_specs=[
            pl.BlockSpec((1, _C, _RT), lambda b, t: (b, 0, t)),
            pl.BlockSpec((1, _C, _N), lambda b, t: (b, 0, 0)),
            pl.BlockSpec((1, 1, _N), lambda b, t: (b, 0, 0)),
        ],
        out_specs=pl.BlockSpec((1, _RT, _K), lambda b, t: (b, t, 0)),
        out_shape=jax.ShapeDtypeStruct((_B, _N, _K), jnp.int32),
        interpret=interpret,
    )(x, x, xx)


_NW = 32
_RPW = _E // _NW
_CH = 256
_NCH = _RPW // _CH


_PPW = _PTS // _NW
_CHP = _CH // _K


def _sc_gather(table, idx2):
    mesh = plsc.VectorSubcoreMesh(core_axis_name="c", subcore_axis_name="s")

    @functools.partial(
        pl.kernel,
        mesh=mesh,
        out_type=jax.ShapeDtypeStruct((_E, _W), jnp.float32),
        scratch_types=[
            pltpu.VMEM((_PPW, _K), jnp.int32),
            pltpu.VMEM((_RPW,), jnp.int32),
            pltpu.VMEM((2, _CH, _W), jnp.float32),
            pltpu.SemaphoreType.DMA,
            pltpu.SemaphoreType.DMA,
        ],
    )
    def k(table_hbm, idx_hbm, out_hbm, idx_a, idx_b, bufs, sem0, sem1):
        wid = lax.axis_index("s") * 2 + lax.axis_index("c")
        pbase = wid * _PPW
        base = wid * _RPW
        pltpu.sync_copy(idx_hbm.at[pl.ds(pbase, _PPW)], idx_a)

        def reorder(t, _):
            idx_b[pl.ds(t * _K, _K)] = idx_a[t, :]
            return 0

        lax.fori_loop(0, _PPW, reorder, 0)
        sems = (sem0, sem1)

        def start(i):
            return pltpu.async_copy(
                table_hbm.at[idx_b.at[pl.ds(i * _CH, _CH)]],
                bufs.at[i % 2], sems[i % 2])

        cps = {0: start(0)}
        for i in range(_NCH):
            if i + 1 < _NCH:
                cps[i + 1] = start(i + 1)
            cps[i].wait()
            pltpu.sync_copy(bufs.at[i % 2],
                            out_hbm.at[pl.ds(base + i * _CH, _CH)])

    return k(table, idx2)


def _stats_body(g_ref, yt_ref, at_ref, p_ref, o_ref):
    g = g_ref[...]
    ybp = _dot(yt_ref[...], p_ref[...], ((1,), (0,)))
    gs = _bdot(at_ref[...], g, ((0,), (0,)))
    o_ref[0] = jnp.concatenate([
        jnp.sum(g, axis=0, keepdims=True),
        jnp.sum(g * g, axis=0, keepdims=True),
        jnp.sum(gs * ybp, axis=0, keepdims=True),
        jnp.sum(ybp, axis=0, keepdims=True),
        jnp.sum(ybp * ybp, axis=0, keepdims=True),
    ], axis=0)


def _stats(g, yt, at, p, interpret=False):
    return pl.pallas_call(
        _stats_body,
        grid=(_NT,),
        in_specs=[
            pl.BlockSpec((_ET, _W), lambda t: (t, 0)),
            pl.BlockSpec((_PT, _W), lambda t: (t, 0)),
            pl.BlockSpec((_ET, _PT), lambda t: (0, 0)),
            pl.BlockSpec((_W, _W), lambda t: (0, 0)),
        ],
        out_specs=pl.BlockSpec((1, 5, _W), lambda t: (t, 0, 0)),
        out_shape=jax.ShapeDtypeStruct((_NT, 5, _W), jnp.float32),
        interpret=interpret,
    )(g, yt, at, p)


def _mid_body(g_ref, yt_ref, at_ref, p_ref, s1_ref, t1_ref, s_ref, sr_ref):
    ybp = _dot(yt_ref[...], p_ref[...], ((1,), (0,)))
    ybx = _bdot(at_ref[...], ybp, ((1,), (0,)))
    h = g_ref[...] + ybx
    r = jnp.maximum(h * s1_ref[...] + t1_ref[...], 0.0)
    s_ref[0] = _bdot(r, r, ((0,), (0,)))
    sr_ref[0] = jnp.sum(r, axis=0, keepdims=True)


def _mid(g, yt, at, p, s1, t1, interpret=False):
    return pl.pallas_call(
        _mid_body,
        grid=(_NT,),
        in_specs=[
            pl.BlockSpec((_ET, _W), lambda t: (t, 0)),
            pl.BlockSpec((_PT, _W), lambda t: (t, 0)),
            pl.BlockSpec((_ET, _PT), lambda t: (0, 0)),
            pl.BlockSpec((_W, _W), lambda t: (0, 0)),
            pl.BlockSpec((1, _W), lambda t: (0, 0)),
            pl.BlockSpec((1, _W), lambda t: (0, 0)),
        ],
        out_specs=[
            pl.BlockSpec((1, _W, _W), lambda t: (t, 0, 0)),
            pl.BlockSpec((1, 1, _W), lambda t: (t, 0, 0)),
        ],
        out_shape=[
            jax.ShapeDtypeStruct((_NT, _W, _W), jnp.float32),
            jax.ShapeDtypeStruct((_NT, 1, _W), jnp.float32),
        ],
        interpret=interpret,
    )(g, yt, at, p, s1, t1)


def _bn2_body(s_ref, sr_ref, w2_ref, b2_ref, g2_ref, be2_ref,
              sc_ref, sh_ref):
    w2 = w2_ref[...]
    m = _dot(w2, s_ref[...], ((1,), (0,)))
    diag = jnp.sum(m * w2, axis=1, keepdims=True)
    wsr = _dot(w2, sr_ref[...], ((1,), (1,)))
    b2 = b2_ref[...]
    inv_e = 1.0 / _E
    mean2 = wsr * inv_e + b2
    ex2 = diag * inv_e + 2.0 * b2 * wsr * inv_e + b2 * b2
    var2 = ex2 - mean2 * mean2
    sc2 = g2_ref[...] / jnp.sqrt(var2 + _EPS)
    sc_ref[...] = sc2
    sh_ref[...] = be2_ref[...] - mean2 * sc2 + sc2 * b2


def _bn2(s, sr, w2, b2c, g2c, be2c, interpret=False):
    full = lambda shp: pl.BlockSpec(shp, lambda: (0,) * len(shp))
    return pl.pallas_call(
        _bn2_body,
        grid=(),
        in_specs=[full((_DH, _DH)), full((1, _DH)), full((_DO, _DH)),
                  full((_DO, 1)), full((_DO, 1)), full((_DO, 1))],
        out_specs=[full((_DO, 1)), full((_DO, 1))],
        out_shape=[jax.ShapeDtypeStruct((_DO, 1), jnp.float32),
                   jax.ShapeDtypeStruct((_DO, 1), jnp.float32)],
        interpret=interpret,
    )(s, sr, w2, b2c, g2c, be2c)


def _final_body(g_ref, yt_ref, at_ref, p_ref, w2_ref, s1_ref, t1_ref,
                s2_ref, t2_ref, o_ref):
    ybp = _dot(yt_ref[...], p_ref[...], ((1,), (0,)))
    ybx = _bdot(at_ref[...], ybp, ((1,), (0,)))
    h = g_ref[...] + ybx
    r = jnp.maximum(h * s1_ref[...] + t1_ref[...], 0.0)
    h2 = _bdot(w2_ref[...], r, ((1,), (1,)))
    o_ref[0] = jnp.maximum(h2 * s2_ref[...] + t2_ref[...], 0.0)


def _final(g, yt, at, p, w2p, s1, t1, s2, t2, interpret=False):
    return pl.pallas_call(
        _final_body,
        grid=(_NT,),
        in_specs=[
            pl.BlockSpec((_ET, _W), lambda t: (t, 0)),
            pl.BlockSpec((_PT, _W), lambda t: (t, 0)),
            pl.BlockSpec((_ET, _PT), lambda t: (0, 0)),
            pl.BlockSpec((_W, _W), lambda t: (0, 0)),
            pl.BlockSpec((_DO, _W), lambda t: (0, 0)),
            pl.BlockSpec((1, _W), lambda t: (0, 0)),
            pl.BlockSpec((1, _W), lambda t: (0, 0)),
            pl.BlockSpec((_DO, 1), lambda t: (0, 0)),
            pl.BlockSpec((_DO, 1), lambda t: (0, 0)),
        ],
        out_specs=pl.BlockSpec((1, _DO, _ET),
                               lambda t: (t // (_N * _K // _ET), 0,
                                          t % (_N * _K // _ET))),
        out_shape=jax.ShapeDtypeStruct((_B, _DO, _N * _K), jnp.float32),
        interpret=interpret,
    )(g, yt, at, p, w2p, s1, t1, s2, t2)


def kernel(x, W1, b1, g1, be1, W2, b2, g2, be2):
    wa = W1[:, :_C]
    w = jnp.concatenate([wa, W1[:, _C:] - wa], axis=0)
    bv = jnp.concatenate([jnp.zeros((_DH,), jnp.float32), b1])[None, :]
    at = jnp.asarray(_AT)
    p = jnp.asarray(_P)

    yt, xx = _pre(x, w, bv)
    idx = _knn(x, xx)

    table = yt.reshape(_PTS, _W)
    g = _sc_gather(table, idx.reshape(_PTS, _K))

    st = jnp.sum(_stats(g, table, at, p), axis=0)[:, :_DH]
    mean1 = (st[0] + float(_K) * st[3]) / _E
    ex2 = (st[1] + 2.0 * st[2] + float(_K) * st[4]) / _E
    var1 = ex2 - mean1 * mean1
    sc1 = g1 / jnp.sqrt(var1 + _EPS)
    sh1 = be1 - mean1 * sc1
    zpad = jnp.zeros((_DH,), jnp.float32)
    s1 = jnp.concatenate([sc1, zpad])[None, :]
    t1 = jnp.concatenate([sh1, zpad])[None, :]

    sp, srp = _mid(g, table, at, p, s1, t1)
    s = jnp.sum(sp, axis=0)[:_DH, :_DH]
    sr = jnp.sum(srp, axis=0)[:, :_DH]

    sc2, sh2 = _bn2(s, sr, W2, b2[:, None], g2[:, None], be2[:, None])

    w2p = jnp.concatenate([W2, jnp.zeros((_DO, _DH), jnp.float32)], axis=1)
    out = _final(g, table, at, p, w2p, s1, t1, sc2, sh2)
    return out.reshape(_B, _DO, _N, _K)
